# Initial kernel scaffold; baseline (speedup 1.0000x reference)
import jax
import jax.numpy as jnp
from jax.experimental import pallas as pl


def kernel(inputs, embedding):
    raise NotImplementedError("write your pallas kernel here")


# speedup vs baseline: 1.0487x; 1.0487x over previous
import functools

import jax
import jax.numpy as jnp
from jax import lax
from jax.experimental import pallas as pl
from jax.experimental.pallas import tpu as pltpu
from jax.experimental.pallas import tpu_sc as plsc

_N_EMB = 1024
_DIM = 64
_ROWS = 16 * 1024
_BLK = 1024
_GRID = _ROWS // _BLK
_LOSS_SCALE = 2.0 / (_ROWS * _DIM)


def _tc_body(x_ref, e_ref, x2_ref, e2_ref, idx_ref, loss_ref, acc_ref):
    i = pl.program_id(0)
    x = x_ref[...]
    e = e_ref[...]
    m = lax.dot_general(x, e, (((1,), (1,)), ((), ())),
                        preferred_element_type=jnp.float32)
    d = (x2_ref[...] + e2_ref[...]) - 2.0 * m
    dmin = jnp.min(d, axis=1, keepdims=True)
    col = lax.broadcasted_iota(jnp.int32, d.shape, 1)
    idx = jnp.min(jnp.where(d == dmin, col, _N_EMB), axis=1)
    idx_ref[0, 0, :] = idx

    @pl.when(i == 0)
    def _():
        acc_ref[0] = 0.0

    acc_ref[0] += jnp.sum(dmin)

    @pl.when(i == _GRID - 1)
    def _():
        loss_ref[0, 0] = acc_ref[0] * _LOSS_SCALE


def _tc_call(flat, embedding, x2, e2, interpret=False):
    return pl.pallas_call(
        _tc_body,
        grid=(_GRID,),
        in_specs=[
            pl.BlockSpec((_BLK, _DIM), lambda i: (i, 0)),
            pl.BlockSpec((_N_EMB, _DIM), lambda i: (0, 0)),
            pl.BlockSpec((_BLK, 1), lambda i: (i, 0)),
            pl.BlockSpec((1, _N_EMB), lambda i: (0, 0)),
        ],
        out_specs=[
            pl.BlockSpec((1, 1, _BLK), lambda i: (i, 0, 0)),
            pl.BlockSpec(memory_space=pltpu.SMEM, block_shape=(1, 1),
                         index_map=lambda i: (0, 0)),
        ],
        out_shape=[
            jax.ShapeDtypeStruct((_GRID, 1, _BLK), jnp.int32),
            jax.ShapeDtypeStruct((1, 1), jnp.float32),
        ],
        scratch_shapes=[pltpu.SMEM((1,), jnp.float32)],
        interpret=interpret,
    )(flat, embedding, x2, e2)


_NW = 32
_BPW = _ROWS // _NW
_CHUNK = 128
_NCHUNK = _BPW // _CHUNK


def _sc_gather(embedding, idx):
    mesh = plsc.VectorSubcoreMesh(core_axis_name="c", subcore_axis_name="s")

    @functools.partial(
        pl.kernel,
        mesh=mesh,
        out_type=jax.ShapeDtypeStruct((_ROWS, _DIM), jnp.float32),
        compiler_params=pltpu.CompilerParams(use_tc_tiling_on_sc=False),
        scratch_types=[
            pltpu.VMEM((_NCHUNK, _CHUNK), jnp.int32),
            pltpu.VMEM((_BPW, _DIM), jnp.float32),
            pltpu.SemaphoreType.DMA,
        ],
    )
    def k(table_hbm, idx_hbm, out_hbm, idx_v, rows_v, sem):
        wid = lax.axis_index("s") * 2 + lax.axis_index("c")
        base = wid * _BPW
        for c in range(_NCHUNK):
            pltpu.sync_copy(idx_hbm.at[pl.ds(base + c * _CHUNK, _CHUNK)],
                            idx_v.at[c])
        copies = []
        for c in range(_NCHUNK):
            copies.append(pltpu.async_copy(
                table_hbm.at[idx_v.at[c]],
                rows_v.at[pl.ds(c * _CHUNK, _CHUNK)], sem))
        for cp in copies:
            cp.wait()
        pltpu.sync_copy(rows_v, out_hbm.at[pl.ds(base, _BPW)])

    return k(embedding, idx)


def kernel(inputs, embedding):
    flat = inputs.reshape(_ROWS, _DIM)
    x2 = jnp.sum(flat ** 2, axis=1, keepdims=True)
    e2 = jnp.sum(embedding ** 2, axis=1).reshape(1, _N_EMB)
    idx2d, loss = _tc_call(flat, embedding, x2, e2)
    idx = idx2d.reshape(_ROWS)
    quantized = _sc_gather(embedding, idx)
    return (quantized.reshape(inputs.shape), loss[0, 0],
            idx[:, None])


# speedup vs baseline: 1.1077x; 1.0562x over previous
import functools

import jax
import jax.numpy as jnp
from jax import lax
from jax.experimental import pallas as pl
from jax.experimental.pallas import tpu as pltpu
from jax.experimental.pallas import tpu_sc as plsc

_N_EMB = 1024
_DIM = 64
_ROWS = 16 * 1024
_BLK = 512
_GRID = _ROWS // _BLK
_CCHUNK = 128
_NCC = _N_EMB // _CCHUNK
_LOSS_SCALE = 2.0 / (_ROWS * _DIM)


def _tc_body(x_ref, e_ref, x2_ref, e2_ref, idx_ref, loss_ref, m_ref, acc_ref):
    i = pl.program_id(0)
    x = x_ref[...]
    e = e_ref[...]
    m_ref[...] = lax.dot_general(x + x, e, (((1,), (1,)), ((), ())),
                                 preferred_element_type=jnp.float32)
    x2 = x2_ref[...]
    lane = lax.broadcasted_iota(jnp.int32, (_BLK, _CCHUNK), 1)
    best = None
    for c in range(_NCC):
        e2c = e2_ref[:, c * _CCHUNK:(c + 1) * _CCHUNK]
        dc = (x2 + e2c) - m_ref[:, c * _CCHUNK:(c + 1) * _CCHUNK]
        colc = lane + (c * _CCHUNK)
        if best is None:
            best, bidx = dc, colc
        else:
            lt = dc < best
            best = jnp.where(lt, dc, best)
            bidx = jnp.where(lt, colc, bidx)
    rowmin = jnp.min(best, axis=1, keepdims=True)
    idx = jnp.min(jnp.where(best == rowmin, bidx, _N_EMB), axis=1)
    idx_ref[0, 0, :] = idx

    @pl.when(i == 0)
    def _():
        acc_ref[0] = 0.0

    acc_ref[0] += jnp.sum(rowmin)

    @pl.when(i == _GRID - 1)
    def _():
        loss_ref[0, 0] = acc_ref[0] * _LOSS_SCALE


def _tc_call(flat, embedding, x2, e2, interpret=False):
    return pl.pallas_call(
        _tc_body,
        grid=(_GRID,),
        in_specs=[
            pl.BlockSpec((_BLK, _DIM), lambda i: (i, 0)),
            pl.BlockSpec((_N_EMB, _DIM), lambda i: (0, 0)),
            pl.BlockSpec((_BLK, 1), lambda i: (i, 0)),
            pl.BlockSpec((1, _N_EMB), lambda i: (0, 0)),
        ],
        out_specs=[
            pl.BlockSpec((1, 1, _BLK), lambda i: (i, 0, 0)),
            pl.BlockSpec(memory_space=pltpu.SMEM, block_shape=(1, 1),
                         index_map=lambda i: (0, 0)),
        ],
        out_shape=[
            jax.ShapeDtypeStruct((_GRID, 1, _BLK), jnp.int32),
            jax.ShapeDtypeStruct((1, 1), jnp.float32),
        ],
        scratch_shapes=[pltpu.VMEM((_BLK, _N_EMB), jnp.float32),
                        pltpu.SMEM((1,), jnp.float32)],
        interpret=interpret,
    )(flat, embedding, x2, e2)


_NW = 32
_BPW = _ROWS // _NW
_CHUNK = 128
_NCHUNK = _BPW // _CHUNK


def _sc_gather(embedding, idx):
    mesh = plsc.VectorSubcoreMesh(core_axis_name="c", subcore_axis_name="s")

    @functools.partial(
        pl.kernel,
        mesh=mesh,
        out_type=jax.ShapeDtypeStruct((_ROWS, _DIM), jnp.float32),
        compiler_params=pltpu.CompilerParams(use_tc_tiling_on_sc=False),
        scratch_types=[
            pltpu.VMEM((_NCHUNK, _CHUNK), jnp.int32),
            pltpu.VMEM((_BPW, _DIM), jnp.float32),
            pltpu.SemaphoreType.DMA,
        ],
    )
    def k(table_hbm, idx_hbm, out_hbm, idx_v, rows_v, sem):
        wid = lax.axis_index("s") * 2 + lax.axis_index("c")
        base = wid * _BPW
        for c in range(_NCHUNK):
            pltpu.sync_copy(idx_hbm.at[pl.ds(base + c * _CHUNK, _CHUNK)],
                            idx_v.at[c])
        copies = []
        for c in range(_NCHUNK):
            copies.append(pltpu.async_copy(
                table_hbm.at[idx_v.at[c]],
                rows_v.at[pl.ds(c * _CHUNK, _CHUNK)], sem))
        for cp in copies:
            cp.wait()
        pltpu.sync_copy(rows_v, out_hbm.at[pl.ds(base, _BPW)])

    return k(embedding, idx)


def kernel(inputs, embedding):
    flat = inputs.reshape(_ROWS, _DIM)
    x2 = jnp.sum(flat ** 2, axis=1, keepdims=True)
    e2 = jnp.sum(embedding ** 2, axis=1).reshape(1, _N_EMB)
    idx2d, loss = _tc_call(flat, embedding, x2, e2)
    idx = idx2d.reshape(_ROWS)
    quantized = _sc_gather(embedding, idx)
    return (quantized.reshape(inputs.shape), loss[0, 0],
            idx[:, None])


# speedup vs baseline: 1.3114x; 1.1839x over previous
import functools

import jax
import jax.numpy as jnp
from jax import lax
from jax.experimental import pallas as pl
from jax.experimental.pallas import tpu as pltpu
from jax.experimental.pallas import tpu_sc as plsc

_N_EMB = 1024
_DIM = 64
_NB = 16
_ROWS = _NB * 1024
_BLK = 512
_GRID = _ROWS // _BLK
_CCHUNK = 128
_NCC = _N_EMB // _CCHUNK
_LOSS_SCALE = 2.0 / (_ROWS * _DIM)


def _tc_body(xt_ref, e_ref, x2_ref, e2_ref, idx_ref, loss_ref, m_ref, acc_ref):
    i = pl.program_id(0)
    xt = xt_ref[0]
    e = e_ref[...]
    m_ref[...] = lax.dot_general(e + e, xt, (((1,), (0,)), ((), ())),
                                 preferred_element_type=jnp.float32)
    x2 = x2_ref[...]
    srow = lax.broadcasted_iota(jnp.int32, (_CCHUNK, _BLK), 0)
    best = None
    for c in range(_NCC):
        sl = pl.ds(c * _CCHUNK, _CCHUNK)
        dc = (x2 + e2_ref[sl, :]) - m_ref[sl, :]
        codec = srow + (c * _CCHUNK)
        if best is None:
            best, bidx = dc, codec
        else:
            lt = dc < best
            best = jnp.where(lt, dc, best)
            bidx = jnp.where(lt, codec, bidx)
    rowmin = jnp.min(best, axis=0, keepdims=True)
    idx = jnp.min(jnp.where(best == rowmin, bidx, _N_EMB), axis=0)
    idx_ref[0, 0, :] = idx

    @pl.when(i == 0)
    def _():
        acc_ref[0] = 0.0

    acc_ref[0] += jnp.sum(rowmin)

    @pl.when(i == _GRID - 1)
    def _():
        loss_ref[0, 0] = acc_ref[0] * _LOSS_SCALE


def _tc_call(xt, embedding, x2, e2, interpret=False):
    nsub = 1024 // _BLK
    return pl.pallas_call(
        _tc_body,
        grid=(_GRID,),
        in_specs=[
            pl.BlockSpec((1, _DIM, _BLK), lambda i: (i // nsub, 0, i % nsub)),
            pl.BlockSpec((_N_EMB, _DIM), lambda i: (0, 0)),
            pl.BlockSpec((1, _BLK), lambda i: (0, i)),
            pl.BlockSpec((_N_EMB, 1), lambda i: (0, 0)),
        ],
        out_specs=[
            pl.BlockSpec((1, 1, _BLK), lambda i: (i, 0, 0)),
            pl.BlockSpec(memory_space=pltpu.SMEM, block_shape=(1, 1),
                         index_map=lambda i: (0, 0)),
        ],
        out_shape=[
            jax.ShapeDtypeStruct((_GRID, 1, _BLK), jnp.int32),
            jax.ShapeDtypeStruct((1, 1), jnp.float32),
        ],
        scratch_shapes=[pltpu.VMEM((_N_EMB, _BLK), jnp.float32),
                        pltpu.SMEM((1,), jnp.float32)],
        interpret=interpret,
    )(xt, embedding, x2, e2)


_NW = 32
_BPW = _ROWS // _NW
_CHUNK = 128
_NCHUNK = _BPW // _CHUNK


def _sc_gather(embedding, idx):
    mesh = plsc.VectorSubcoreMesh(core_axis_name="c", subcore_axis_name="s")

    @functools.partial(
        pl.kernel,
        mesh=mesh,
        out_type=jax.ShapeDtypeStruct((_NB, 1024, _DIM), jnp.float32),
        compiler_params=pltpu.CompilerParams(use_tc_tiling_on_sc=False),
        scratch_types=[
            pltpu.VMEM((_NCHUNK, _CHUNK), jnp.int32),
            pltpu.VMEM((_BPW, _DIM), jnp.float32),
            pltpu.SemaphoreType.DMA,
        ],
    )
    def k(table_hbm, idx_hbm, out_hbm, idx_v, rows_v, sem):
        wid = lax.axis_index("s") * 2 + lax.axis_index("c")
        base = wid * _BPW
        for c in range(_NCHUNK):
            pltpu.sync_copy(idx_hbm.at[pl.ds(base + c * _CHUNK, _CHUNK)],
                            idx_v.at[c])
        copies = []
        for c in range(_NCHUNK):
            copies.append(pltpu.async_copy(
                table_hbm.at[idx_v.at[c]],
                rows_v.at[pl.ds(c * _CHUNK, _CHUNK)], sem))
        for cp in copies:
            cp.wait()
        b = wid // (1024 // _BPW)
        r0 = (wid % (1024 // _BPW)) * _BPW
        pltpu.sync_copy(rows_v, out_hbm.at[b, pl.ds(r0, _BPW), :])

    return k(embedding, idx)


def kernel(inputs, embedding):
    xt = inputs.transpose(0, 2, 1)
    x2 = jnp.sum(inputs.reshape(_ROWS, _DIM) ** 2, axis=1).reshape(1, _ROWS)
    e2 = jnp.sum(embedding ** 2, axis=1).reshape(_N_EMB, 1)
    idx2d, loss = _tc_call(xt, embedding, x2, e2)
    idx = idx2d.reshape(_ROWS)
    quantized = _sc_gather(embedding, idx)
    return (quantized, loss[0, 0], idx[:, None])
